# initial kernel scaffold (unmeasured)
import functools

import jax
import jax.numpy as jnp
from jax import lax
from jax.experimental import pallas as pl
from jax.experimental.pallas import tpu as pltpu

NZ = 4


def kernel(partial, gamma):
    _, m, d = partial.shape
    ch = m // NZ

    def body(x_ref, g_ref, out_ref, comm_ref, stage_ref, send_sems, recv_sems, copy_sem):
        my_x = lax.axis_index("x")
        my_y = lax.axis_index("y")
        my_z = lax.axis_index("z")
        right = (my_z + 1) % NZ
        left = (my_z - 1) % NZ

        barrier = pltpu.get_barrier_semaphore()
        for nbr in (left, right):
            pl.semaphore_signal(
                barrier, inc=1,
                device_id=(my_x, my_y, nbr),
                device_id_type=pl.DeviceIdType.MESH,
            )
        pl.semaphore_wait(barrier, 2)

        def load_chunk(c):
            cp = pltpu.make_async_copy(
                x_ref.at[0, pl.ds(c * ch, ch), :], stage_ref, copy_sem
            )
            cp.start()
            cp.wait()

        load_chunk((my_z - 1) % NZ)
        comm_ref[3, :, :] = stage_ref[:, :].astype(jnp.bfloat16)

        send_slot = 3
        for h in range(NZ - 1):
            rdma = pltpu.make_async_remote_copy(
                src_ref=comm_ref.at[send_slot],
                dst_ref=comm_ref.at[h],
                send_sem=send_sems.at[h],
                recv_sem=recv_sems.at[h],
                device_id=(my_x, my_y, right),
                device_id_type=pl.DeviceIdType.MESH,
            )
            rdma.start()
            load_chunk((my_z - h - 2) % NZ)
            rdma.wait()
            acc = comm_ref[h, :, :].astype(jnp.float32) + stage_ref[:, :]
            if h < NZ - 2:
                comm_ref[h, :, :] = acc.astype(jnp.bfloat16)
                send_slot = h
            else:
                rms = jnp.sqrt(
                    jnp.mean(acc * acc, axis=-1, keepdims=True) + 1e-6
                )
                out_ref[:, :] = acc / rms * g_ref[0, :][None, :]

        @functools.partial(pl.run_scoped, sem=pltpu.SemaphoreType.REGULAR)
        def _(sem):
            for nbr in (left, right):
                pl.semaphore_signal(
                    sem, inc=1,
                    device_id=(my_x, my_y, nbr),
                    device_id_type=pl.DeviceIdType.MESH,
                )
            pl.semaphore_wait(sem, 2)

    return pl.pallas_call(
        body,
        out_shape=jax.ShapeDtypeStruct((ch, d), jnp.float32),
        in_specs=[
            pl.BlockSpec(memory_space=pltpu.ANY),
            pl.BlockSpec(memory_space=pltpu.VMEM),
        ],
        out_specs=pl.BlockSpec(memory_space=pltpu.VMEM),
        scratch_shapes=[
            pltpu.VMEM((NZ, ch, d), jnp.bfloat16),
            pltpu.VMEM((ch, d), jnp.float32),
            pltpu.SemaphoreType.DMA((NZ - 1,)),
            pltpu.SemaphoreType.DMA((NZ - 1,)),
            pltpu.SemaphoreType.DMA,
        ],
        compiler_params=pltpu.CompilerParams(collective_id=0),
    )(partial, gamma.reshape(1, d))


# baseline (device time: 314103 ns/iter reference)
import functools

import jax
import jax.numpy as jnp
from jax import lax
from jax.experimental import pallas as pl
from jax.experimental.pallas import tpu as pltpu

NZ = 4
TILE = 512


def kernel(partial, gamma):
    _, m, d = partial.shape
    ch = m // NZ

    def body(x_ref, g_ref, out_ref, comm_ref, send_sems, recv_sems, copy_sem,
             credit_sem):
        my_x = lax.axis_index("x")
        my_y = lax.axis_index("y")
        my_z = lax.axis_index("z")
        right = (my_z + 1) % NZ
        left = (my_z - 1) % NZ

        barrier = pltpu.get_barrier_semaphore()
        for nbr in (left, right):
            pl.semaphore_signal(
                barrier, inc=1,
                device_id=(my_x, my_y, nbr),
                device_id_type=pl.DeviceIdType.MESH,
            )
        pl.semaphore_wait(barrier, 2)

        def load_chunk(c):
            cp = pltpu.make_async_copy(
                x_ref.at[0, pl.ds(c * ch, ch), :], out_ref, copy_sem
            )
            cp.start()
            cp.wait()

        load_chunk((my_z - 1) % NZ)
        for t in range(0, ch, TILE):
            comm_ref[2, pl.ds(t, TILE), :] = (
                out_ref[pl.ds(t, TILE), :].astype(jnp.bfloat16)
            )

        send_slot = 2
        for h in range(NZ - 1):
            if h == NZ - 2:
                pl.semaphore_wait(credit_sem, 1)
            rdma = pltpu.make_async_remote_copy(
                src_ref=comm_ref.at[send_slot],
                dst_ref=comm_ref.at[h],
                send_sem=send_sems.at[h],
                recv_sem=recv_sems.at[h],
                device_id=(my_x, my_y, right),
                device_id_type=pl.DeviceIdType.MESH,
            )
            rdma.start()
            load_chunk((my_z - h - 2) % NZ)
            rdma.wait()
            if h == 0:
                pl.semaphore_signal(
                    credit_sem, inc=1,
                    device_id=(my_x, my_y, left),
                    device_id_type=pl.DeviceIdType.MESH,
                )
            if h < NZ - 2:
                for t in range(0, ch, TILE):
                    comm_ref[h, pl.ds(t, TILE), :] = (
                        comm_ref[h, pl.ds(t, TILE), :].astype(jnp.float32)
                        + out_ref[pl.ds(t, TILE), :]
                    ).astype(jnp.bfloat16)
                send_slot = h
            else:
                g = g_ref[0, :][None, :]
                for t in range(0, ch, TILE):
                    acc = (
                        comm_ref[h, pl.ds(t, TILE), :].astype(jnp.float32)
                        + out_ref[pl.ds(t, TILE), :]
                    )
                    rms = jnp.sqrt(
                        jnp.mean(acc * acc, axis=-1, keepdims=True) + 1e-6
                    )
                    out_ref[pl.ds(t, TILE), :] = acc / rms * g

        @functools.partial(pl.run_scoped, sem=pltpu.SemaphoreType.REGULAR)
        def _(sem):
            for nbr in (left, right):
                pl.semaphore_signal(
                    sem, inc=1,
                    device_id=(my_x, my_y, nbr),
                    device_id_type=pl.DeviceIdType.MESH,
                )
            pl.semaphore_wait(sem, 2)

    return pl.pallas_call(
        body,
        out_shape=jax.ShapeDtypeStruct((ch, d), jnp.float32),
        in_specs=[
            pl.BlockSpec(memory_space=pl.ANY),
            pl.BlockSpec(memory_space=pltpu.VMEM),
        ],
        out_specs=pl.BlockSpec(memory_space=pltpu.VMEM),
        scratch_shapes=[
            pltpu.VMEM((NZ - 1, ch, d), jnp.bfloat16),
            pltpu.SemaphoreType.DMA((NZ - 1,)),
            pltpu.SemaphoreType.DMA((NZ - 1,)),
            pltpu.SemaphoreType.DMA,
            pltpu.SemaphoreType.REGULAR,
        ],
        compiler_params=pltpu.CompilerParams(
            collective_id=0,
            vmem_limit_bytes=63 * 1024 * 1024,
        ),
    )(partial, gamma.reshape(1, d))
